# Initial kernel scaffold; baseline (speedup 1.0000x reference)
import jax
import jax.numpy as jnp
from jax.experimental import pallas as pl


def kernel(x, edge_index, batch, W_l1, W_r1, W_l2, W_r2, W_p1, b_p1, W_p2, b_p2):
    raise NotImplementedError("write your pallas kernel here")


# speedup vs baseline: 6.4526x; 6.4526x over previous
import functools

import jax
import jax.numpy as jnp
from jax import lax
from jax.experimental import pallas as pl
from jax.experimental.pallas import tpu as pltpu
from jax.experimental.pallas import tpu_sc as plsc

NC = 2
NS = 16
NW = NC * NS


def _make_agg(n_pad, d, nchunks, k, with_counts):
  n = n_pad
  stripe = n // NS
  del with_counts
  mesh = plsc.VectorSubcoreMesh(core_axis_name="c", subcore_axis_name="s")
  out_type = [jax.ShapeDtypeStruct((NC, n, d), jnp.float32)]
  scratch = [
      pltpu.VMEM((nchunks, k), jnp.int32),
      pltpu.VMEM((nchunks, k), jnp.int32),
      pltpu.VMEM((k, d), jnp.float32),
      pltpu.VMEM_SHARED((n, d), jnp.float32),
      pltpu.SemaphoreType.DMA,
  ]

  def body(x_hbm, src_hbm, dst_hbm, zrow_hbm, sums_hbm,
           idx_s, idx_d, rows, accum, sem):
    c = lax.axis_index("c")
    s = lax.axis_index("s")
    base = s * stripe
    pltpu.sync_copy(zrow_hbm, accum.at[pl.ds(base, stripe)])
    pltpu.sync_copy(src_hbm.at[c, s], idx_s)
    pltpu.sync_copy(dst_hbm.at[c, s], idx_d)
    plsc.subcore_barrier()

    def step(j, carry):
      pltpu.async_copy(x_hbm.at[idx_s.at[j]], rows, sem).wait()
      pltpu.sync_copy(rows, accum.at[idx_d.at[j]], add=True)
      return carry

    lax.fori_loop(0, nchunks, step, 0)
    plsc.subcore_barrier()
    pltpu.sync_copy(accum.at[pl.ds(base, stripe)],
                    sums_hbm.at[c, pl.ds(base, stripe)])

  return pl.kernel(body, out_type=out_type, mesh=mesh, scratch_types=scratch)


def _make_counts(n_pad, ngroups):
  mesh = plsc.VectorSubcoreMesh(core_axis_name="c", subcore_axis_name="s")
  nr = n_pad // 128
  out_type = [jax.ShapeDtypeStruct((NC, NS, nr, 128), jnp.float32)]
  scratch = [
      pltpu.VMEM((ngroups, 16), jnp.int32),
      pltpu.VMEM((nr, 128), jnp.float32),
  ]

  def body(dst_hbm, zn_hbm, cnt_hbm, idx_d, hist):
    c = lax.axis_index("c")
    s = lax.axis_index("s")
    pltpu.sync_copy(zn_hbm, hist)
    pltpu.sync_copy(dst_hbm.at[c, s], idx_d)
    ones = jnp.full((16,), 1.0, jnp.float32)

    def step(j, carry):
      idx = idx_d[j]
      row = lax.shift_right_logical(idx, 7)
      col = lax.bitwise_and(idx, 127)
      plsc.addupdate_scatter(hist, [row, col], ones)
      return carry

    lax.fori_loop(0, ngroups, step, 0)
    pltpu.sync_copy(hist, cnt_hbm.at[c, s])

  return pl.kernel(
      body, out_type=out_type, mesh=mesh, scratch_types=scratch,
      compiler_params=pltpu.CompilerParams(needs_layout_passes=False))


def _dotT(a, w):
  return lax.dot_general(a, w, (((1,), (1,)), ((), ())),
                         preferred_element_type=jnp.float32)


def _tc1_body(x_ref, s_ref, c_ref, wl_ref, wr_ref, o_ref):
  cnt = jnp.sum(c_ref[...], axis=(0, 1))[:, None]
  agg = (s_ref[0] + s_ref[1]) / jnp.maximum(cnt, 1.0)
  out = _dotT(x_ref[...], wl_ref[...]) + _dotT(agg, wr_ref[...])
  nrm = jnp.sqrt(jnp.sum(out * out, axis=1, keepdims=True))
  out = out / jnp.maximum(nrm, 1e-12)
  o_ref[...] = jnp.maximum(out, 0.0)


def _tc2_body(x_ref, s_ref, c_ref, wl_ref, wr_ref,
              wp1_ref, bp1_ref, wp2_ref, bp2_ref, o_ref):
  cnt = jnp.sum(c_ref[...], axis=(0, 1))[:, None]
  agg = (s_ref[0] + s_ref[1]) / jnp.maximum(cnt, 1.0)
  out = _dotT(x_ref[...], wl_ref[...]) + _dotT(agg, wr_ref[...])
  nrm = jnp.sqrt(jnp.sum(out * out, axis=1, keepdims=True))
  out = out / jnp.maximum(nrm, 1e-12)
  out = jnp.maximum(out, 0.0)
  out = _dotT(out, wp1_ref[...]) + bp1_ref[...]
  out = _dotT(out, wp2_ref[...]) + bp2_ref[...]
  o_ref[...] = out


def kernel(x, edge_index, batch, W_l1, W_r1, W_l2, W_r2, W_p1, b_p1, W_p2,
           b_p2):
  n, d = x.shape
  e = edge_index.shape[1]
  ew = e // NW
  k = 80
  nchunks = ew // k
  assert ew * NW == e and nchunks * k == ew
  n_pad = -(-n // (NS * 8)) * (NS * 8)
  stripe = n_pad // NS

  src_r = edge_index[0].reshape(NC, NS, nchunks, k)
  dst_r = edge_index[1].reshape(NC, NS, nchunks, k)
  dst_g = edge_index[1].reshape(NC, NS, ew // 16, 16)
  zrow = jnp.zeros((stripe, d), jnp.float32)
  zn = jnp.zeros((n_pad // 128, 128), jnp.float32)

  (cnt,) = _make_counts(n_pad, ew // 16)(dst_g, zn)
  cnt = cnt.reshape(NC, NS, n_pad)
  agg_a = _make_agg(n_pad, d, nchunks, k, False)
  (sums1,) = agg_a(x, src_r, dst_r, zrow)

  x_p = jnp.pad(x, ((0, n_pad - n), (0, 0)))
  bn = 128
  grid = (n_pad // bn,)
  row_spec = pl.BlockSpec((bn, d), lambda i: (i, 0))
  sum_spec = pl.BlockSpec((NC, bn, d), lambda i: (0, i, 0))
  cnt_spec = pl.BlockSpec((NC, NS, bn), lambda i: (0, 0, i))
  w_spec = pl.BlockSpec((d, d), lambda i: (0, 0))
  b_spec = pl.BlockSpec((1, d), lambda i: (0, 0))

  h1 = pl.pallas_call(
      _tc1_body,
      grid=grid,
      in_specs=[row_spec, sum_spec, cnt_spec, w_spec, w_spec],
      out_specs=row_spec,
      out_shape=jax.ShapeDtypeStruct((n_pad, d), jnp.float32),
  )(x_p, sums1, cnt, W_l1, W_r1)

  agg_b = _make_agg(n_pad, d, nchunks, k, False)
  (sums2,) = agg_b(h1, src_r, dst_r, zrow)

  out = pl.pallas_call(
      _tc2_body,
      grid=grid,
      in_specs=[row_spec, sum_spec, cnt_spec, w_spec, w_spec,
                w_spec, b_spec, w_spec, b_spec],
      out_specs=row_spec,
      out_shape=jax.ShapeDtypeStruct((n_pad, d), jnp.float32),
  )(h1, sums2, cnt, W_l2, W_r2, W_p1, b_p1.reshape(1, d), W_p2,
    b_p2.reshape(1, d))
  return out[:n]
